# baseline (speedup 1.0000x reference)
import functools

import jax
import jax.numpy as jnp
from jax import lax
from jax.experimental import pallas as pl
from jax.experimental.pallas import tpu as pltpu
from jax.experimental.pallas import tpu_sc as plsc

F = 100000
E = 150000
D = 192
DP = 256
L = 16
NC, NS = 2, 16
NW = NC * NS
B = 128
CHUNKS = 37
E_PER_W = B * CHUNKS
E_PAD = E_PER_W * NW


def _tv_kernel(patch_hbm, idx0_hbm, idx1_hbm, len_hbm, out_hbm,
               i0_v, i1_v, len_v, f1_v, f2_v, acc_v, sem0, sem1, sem2):
    wid = lax.axis_index("s") * NC + lax.axis_index("c")
    base = pl.multiple_of(wid * E_PER_W, B)

    def chunk_body(c, tot):
        off = pl.multiple_of(base + c * B, B)
        cpi0 = pltpu.async_copy(idx0_hbm.at[pl.ds(off, B)], i0_v, sem0)
        cpi1 = pltpu.async_copy(idx1_hbm.at[pl.ds(off, B)], i1_v, sem1)
        cpl = pltpu.async_copy(len_hbm.at[pl.ds(off, B)], len_v, sem2)
        cpi0.wait()
        cpi1.wait()
        cpl.wait()
        cp0 = pltpu.async_copy(patch_hbm.at[i0_v], f1_v, sem0)
        cp1 = pltpu.async_copy(patch_hbm.at[i1_v], f2_v, sem1)
        cp0.wait()
        cp1.wait()

        def group_body(g, t):
            w_blk = len_v[pl.ds(g * L, L)]
            for k in range(L):
                e = g * L + k
                acc = jnp.abs(f1_v[e, 0, pl.ds(0, L)]
                              - f2_v[e, 0, pl.ds(0, L)])
                for j in range(1, D // L):
                    s, o = divmod(j * L, 128)
                    acc += jnp.abs(f1_v[e, s, pl.ds(o, L)]
                                   - f2_v[e, s, pl.ds(o, L)])
                t = t + w_blk[k] * acc
            return t

        return lax.fori_loop(0, B // L, group_body, tot)

    tot = lax.fori_loop(0, CHUNKS, chunk_body, jnp.zeros((L,), jnp.float32))
    acc_v[...] = tot
    pltpu.sync_copy(acc_v, out_hbm.at[pl.ds(wid * L, L)])


@jax.jit
def kernel(adv_patch, face_to_edges_idx, edge_len):
    patch2d = adv_patch.reshape(F, D)
    patch3d = jnp.pad(patch2d, ((0, 0), (0, DP - D))).reshape(F, 2, 128)
    idx = face_to_edges_idx.astype(jnp.int32)
    pad = E_PAD - E
    idx0 = jnp.pad(idx[:, 0], (0, pad))
    idx1 = jnp.pad(idx[:, 1], (0, pad))
    len_p = jnp.pad(edge_len, (0, pad))

    mesh = plsc.VectorSubcoreMesh(core_axis_name="c", subcore_axis_name="s")
    run = pl.kernel(
        _tv_kernel,
        mesh=mesh,
        compiler_params=pltpu.CompilerParams(use_tc_tiling_on_sc=True),
        out_type=jax.ShapeDtypeStruct((NW * L,), jnp.float32),
        scratch_types=[
            pltpu.VMEM((B,), jnp.int32),
            pltpu.VMEM((B,), jnp.int32),
            pltpu.VMEM((B,), jnp.float32),
            pltpu.VMEM((B, 2, 128), jnp.float32),
            pltpu.VMEM((B, 2, 128), jnp.float32),
            pltpu.VMEM((L,), jnp.float32),
            pltpu.SemaphoreType.DMA,
            pltpu.SemaphoreType.DMA,
            pltpu.SemaphoreType.DMA,
        ],
    )
    partials = run(patch3d, idx0, idx1, len_p)
    return jnp.sum(partials) / F


# speedup vs baseline: 1.9017x; 1.9017x over previous
import functools

import jax
import jax.numpy as jnp
from jax import lax
from jax.experimental import pallas as pl
from jax.experimental.pallas import tpu as pltpu
from jax.experimental.pallas import tpu_sc as plsc

F = 100000
E = 150000
D = 192
DP = 256
L = 16
NC, NS = 2, 16
NW = NC * NS
B = 128
CHUNKS = 37
E_PER_W = B * CHUNKS
E_PAD = E_PER_W * NW
BF = 2000


def _pad_rows_kernel(x_ref, o_ref):
    o_ref[:, :D] = x_ref[...]
    o_ref[:, D:] = jnp.zeros((BF, DP - D), jnp.float32)


def _tv_kernel(patch_hbm, idx0_hbm, idx1_hbm, len_hbm, out_hbm,
               i0_v, i1_v, len_v, f1_v, f2_v, acc_v, sem0, sem1, sem2):
    wid = lax.axis_index("s") * NC + lax.axis_index("c")
    base = pl.multiple_of(wid * E_PER_W, B)

    def chunk_body(c, tot):
        off = pl.multiple_of(base + c * B, B)
        cpi0 = pltpu.async_copy(idx0_hbm.at[pl.ds(off, B)], i0_v, sem0)
        cpi1 = pltpu.async_copy(idx1_hbm.at[pl.ds(off, B)], i1_v, sem1)
        cpl = pltpu.async_copy(len_hbm.at[pl.ds(off, B)], len_v, sem2)
        cpi0.wait()
        cpi1.wait()
        cpl.wait()
        cp0 = pltpu.async_copy(patch_hbm.at[i0_v], f1_v, sem0)
        cp1 = pltpu.async_copy(patch_hbm.at[i1_v], f2_v, sem1)
        cp0.wait()
        cp1.wait()

        def group_body(g, t):
            w_blk = len_v[pl.ds(g * L, L)]
            for k in range(L):
                e = g * L + k
                acc = jnp.abs(f1_v[e, pl.ds(0, L)] - f2_v[e, pl.ds(0, L)])
                for j in range(1, D // L):
                    acc += jnp.abs(f1_v[e, pl.ds(j * L, L)]
                                   - f2_v[e, pl.ds(j * L, L)])
                t = t + w_blk[k] * acc
            return t

        return lax.fori_loop(0, B // L, group_body, tot)

    tot = lax.fori_loop(0, CHUNKS, chunk_body, jnp.zeros((L,), jnp.float32))
    acc_v[...] = tot
    pltpu.sync_copy(acc_v, out_hbm.at[pl.ds(wid * L, L)])


@jax.jit
def kernel(adv_patch, face_to_edges_idx, edge_len):
    patch2d = adv_patch.reshape(F, D)
    patch_p = pl.pallas_call(
        _pad_rows_kernel,
        grid=(F // BF,),
        in_specs=[pl.BlockSpec((BF, D), lambda i: (i, 0))],
        out_specs=pl.BlockSpec((BF, DP), lambda i: (i, 0)),
        out_shape=jax.ShapeDtypeStruct((F, DP), jnp.float32),
    )(patch2d)

    idx = face_to_edges_idx.astype(jnp.int32)
    pad = E_PAD - E
    idx0 = jnp.pad(idx[:, 0], (0, pad))
    idx1 = jnp.pad(idx[:, 1], (0, pad))
    len_p = jnp.pad(edge_len, (0, pad))

    mesh = plsc.VectorSubcoreMesh(core_axis_name="c", subcore_axis_name="s")
    run = pl.kernel(
        _tv_kernel,
        mesh=mesh,
        compiler_params=pltpu.CompilerParams(use_tc_tiling_on_sc=True),
        out_type=jax.ShapeDtypeStruct((NW * L,), jnp.float32),
        scratch_types=[
            pltpu.VMEM((B,), jnp.int32),
            pltpu.VMEM((B,), jnp.int32),
            pltpu.VMEM((B,), jnp.float32),
            pltpu.VMEM((B, DP), jnp.float32),
            pltpu.VMEM((B, DP), jnp.float32),
            pltpu.VMEM((L,), jnp.float32),
            pltpu.SemaphoreType.DMA,
            pltpu.SemaphoreType.DMA,
            pltpu.SemaphoreType.DMA,
        ],
    )
    partials = run(patch_p, idx0, idx1, len_p)
    return jnp.sum(partials) / F


# speedup vs baseline: 2.6574x; 1.3974x over previous
import functools

import jax
import jax.numpy as jnp
from jax import lax
from jax.experimental import pallas as pl
from jax.experimental.pallas import tpu as pltpu
from jax.experimental.pallas import tpu_sc as plsc

F = 100000
E = 150000
D = 192
DP = 256
L = 16
NC, NS = 2, 16
NW = NC * NS
B = 112
CHUNKS = 42
E_PER_W = B * CHUNKS
E_PAD = E_PER_W * NW
BF = 2000


def _pad_rows_kernel(x_ref, o_ref):
    o_ref[:, :D] = x_ref[...]
    o_ref[:, D:] = jnp.zeros((BF, DP - D), jnp.float32)


def _tv_kernel(patch_hbm, idx0_hbm, idx1_hbm, len_hbm, out_hbm,
               i0a, i1a, lna, f1a, f2a,
               i0b, i1b, lnb, f1b, f2b,
               acc_v, gsema, gsemb, isem):
    wid = lax.axis_index("s") * NC + lax.axis_index("c")
    base = pl.multiple_of(wid * E_PER_W, B)
    phases = ((i0a, i1a, lna, f1a, f2a, gsema),
              (i0b, i1b, lnb, f1b, f2b, gsemb))

    def load_idx(c, ph):
        i0, i1, ln, _, _, _ = ph
        off = pl.multiple_of(base + c * B, B)
        cp0 = pltpu.async_copy(idx0_hbm.at[pl.ds(off, B)], i0, isem)
        cp1 = pltpu.async_copy(idx1_hbm.at[pl.ds(off, B)], i1, isem)
        cp2 = pltpu.async_copy(len_hbm.at[pl.ds(off, B)], ln, isem)
        cp0.wait()
        cp1.wait()
        cp2.wait()

    def fire_gathers(ph):
        i0, i1, _, f1, f2, gsem = ph
        pltpu.async_copy(patch_hbm.at[i0], f1, gsem)
        pltpu.async_copy(patch_hbm.at[i1], f2, gsem)

    def wait_gathers(ph):
        i0, i1, _, f1, f2, gsem = ph
        pltpu.make_async_copy(patch_hbm.at[i0], f1, gsem).wait()
        pltpu.make_async_copy(patch_hbm.at[i1], f2, gsem).wait()

    def compute(ph, tot):
        _, _, ln, f1, f2, _ = ph

        def group_body(g, t):
            w_blk = ln[pl.ds(g * L, L)]
            for k in range(L):
                e = g * L + k
                acc = jnp.abs(f1[e, pl.ds(0, L)] - f2[e, pl.ds(0, L)])
                for j in range(1, D // L):
                    acc += jnp.abs(f1[e, pl.ds(j * L, L)]
                                   - f2[e, pl.ds(j * L, L)])
                t = t + w_blk[k] * acc
            return t

        return lax.fori_loop(0, B // L, group_body, tot)

    for b in range(2):
        load_idx(b, phases[b])
        fire_gathers(phases[b])

    def pair_body(c2, tot):
        for b in range(2):
            ph = phases[b]
            c = 2 * c2 + b
            wait_gathers(ph)
            tot = compute(ph, tot)

            @pl.when(c + 2 < CHUNKS)
            def _():
                load_idx(c + 2, ph)
                fire_gathers(ph)

        return tot

    tot = lax.fori_loop(0, CHUNKS // 2, pair_body,
                        jnp.zeros((L,), jnp.float32))
    acc_v[...] = tot
    pltpu.sync_copy(acc_v, out_hbm.at[pl.ds(wid * L, L)])


@jax.jit
def kernel(adv_patch, face_to_edges_idx, edge_len):
    patch2d = adv_patch.reshape(F, D)
    patch_p = pl.pallas_call(
        _pad_rows_kernel,
        grid=(F // BF,),
        in_specs=[pl.BlockSpec((BF, D), lambda i: (i, 0))],
        out_specs=pl.BlockSpec((BF, DP), lambda i: (i, 0)),
        out_shape=jax.ShapeDtypeStruct((F, DP), jnp.float32),
    )(patch2d)

    idx = face_to_edges_idx.astype(jnp.int32)
    pad = E_PAD - E
    idx0 = jnp.pad(idx[:, 0], (0, pad))
    idx1 = jnp.pad(idx[:, 1], (0, pad))
    len_p = jnp.pad(edge_len, (0, pad))

    mesh = plsc.VectorSubcoreMesh(core_axis_name="c", subcore_axis_name="s")
    run = pl.kernel(
        _tv_kernel,
        mesh=mesh,
        compiler_params=pltpu.CompilerParams(use_tc_tiling_on_sc=True),
        out_type=jax.ShapeDtypeStruct((NW * L,), jnp.float32),
        scratch_types=[
            pltpu.VMEM((B,), jnp.int32),
            pltpu.VMEM((B,), jnp.int32),
            pltpu.VMEM((B,), jnp.float32),
            pltpu.VMEM((B, DP), jnp.float32),
            pltpu.VMEM((B, DP), jnp.float32),
            pltpu.VMEM((B,), jnp.int32),
            pltpu.VMEM((B,), jnp.int32),
            pltpu.VMEM((B,), jnp.float32),
            pltpu.VMEM((B, DP), jnp.float32),
            pltpu.VMEM((B, DP), jnp.float32),
            pltpu.VMEM((L,), jnp.float32),
            pltpu.SemaphoreType.DMA,
            pltpu.SemaphoreType.DMA,
            pltpu.SemaphoreType.DMA,
        ],
    )
    partials = run(patch_p, idx0, idx1, len_p)
    return jnp.sum(partials) / F


# speedup vs baseline: 3.5948x; 1.3528x over previous
import functools

import jax
import jax.numpy as jnp
from jax import lax
from jax.experimental import pallas as pl
from jax.experimental.pallas import tpu as pltpu
from jax.experimental.pallas import tpu_sc as plsc

F = 100000
E = 150000
D = 192
DP = 256
L = 16
NC, NS = 2, 16
NW = NC * NS
B = 112
CHUNKS = 42
E_PER_W = B * CHUNKS
E_PAD = E_PER_W * NW
BT = 512


def _relayout_kernel(xt_ref, o_ref):
    y = xt_ref[...].T.astype(jnp.bfloat16)
    a = y[:, :128]
    b = jnp.concatenate([y[:, 128:], jnp.zeros((BT, 64), jnp.bfloat16)],
                        axis=1)
    au = jax.lax.bitcast_convert_type(a, jnp.uint16).astype(jnp.int32)
    bu = jax.lax.bitcast_convert_type(b, jnp.uint16).astype(jnp.int32)
    o_ref[...] = au | (bu << 16)


def _tv_kernel(patch_hbm, idx0_hbm, idx1_hbm, len_hbm, out_hbm,
               i0a, i1a, lna, f1a, f2a,
               i0b, i1b, lnb, f1b, f2b,
               acc_v, gsema, gsemb, isem):
    wid = lax.axis_index("s") * NC + lax.axis_index("c")
    base = pl.multiple_of(wid * E_PER_W, B)
    phases = ((i0a, i1a, lna, f1a, f2a, gsema),
              (i0b, i1b, lnb, f1b, f2b, gsemb))

    def load_idx(c, ph):
        i0, i1, ln, _, _, _ = ph
        off = pl.multiple_of(base + c * B, B)
        cp0 = pltpu.async_copy(idx0_hbm.at[pl.ds(off, B)], i0, isem)
        cp1 = pltpu.async_copy(idx1_hbm.at[pl.ds(off, B)], i1, isem)
        cp2 = pltpu.async_copy(len_hbm.at[pl.ds(off, B)], ln, isem)
        cp0.wait()
        cp1.wait()
        cp2.wait()

    def fire_gathers(ph):
        i0, i1, _, f1, f2, gsem = ph
        pltpu.async_copy(patch_hbm.at[i0], f1, gsem)
        pltpu.async_copy(patch_hbm.at[i1], f2, gsem)

    def wait_gathers(ph):
        i0, i1, _, f1, f2, gsem = ph
        pltpu.make_async_copy(patch_hbm.at[i0], f1, gsem).wait()
        pltpu.make_async_copy(patch_hbm.at[i1], f2, gsem).wait()

    def compute(ph, tot):
        _, _, ln, f1, f2, _ = ph

        def group_body(g, t):
            w_blk = ln[pl.ds(g * L, L)]
            for k in range(L):
                e = g * L + k
                facc = None
                for h in range(128 // L):
                    x1 = plsc.bitcast(f1[e, pl.ds(h * L, L)], jnp.bfloat16)
                    x2 = plsc.bitcast(f2[e, pl.ds(h * L, L)], jnp.bfloat16)
                    d = x1 - x2
                    a = jnp.maximum(d, -d)
                    a0, a1 = plsc.unpack(
                        a, format=plsc.PackFormat.INTERLEAVED)
                    facc = a0 + a1 if facc is None else facc + a0 + a1
                t = t + w_blk[k] * facc
            return t

        return lax.fori_loop(0, B // L, group_body, tot)

    for b in range(2):
        load_idx(b, phases[b])
        fire_gathers(phases[b])

    def pair_body(c2, tot):
        for b in range(2):
            ph = phases[b]
            c = 2 * c2 + b
            wait_gathers(ph)
            tot = compute(ph, tot)

            @pl.when(c + 2 < CHUNKS)
            def _():
                load_idx(c + 2, ph)
                fire_gathers(ph)

        return tot

    tot = lax.fori_loop(0, CHUNKS // 2, pair_body,
                        jnp.zeros((L,), jnp.float32))
    acc_v[...] = tot
    pltpu.sync_copy(acc_v, out_hbm.at[pl.ds(wid * L, L)])


@jax.jit
def kernel(adv_patch, face_to_edges_idx, edge_len):
    patch_t = adv_patch.reshape(F, D).T
    patch_p = pl.pallas_call(
        _relayout_kernel,
        grid=(pl.cdiv(F, BT),),
        in_specs=[pl.BlockSpec((D, BT), lambda i: (0, i))],
        out_specs=pl.BlockSpec((BT, 128), lambda i: (i, 0)),
        out_shape=jax.ShapeDtypeStruct((F, 128), jnp.int32),
    )(patch_t)

    idx = face_to_edges_idx.astype(jnp.int32)
    pad = E_PAD - E
    idx0 = jnp.pad(idx[:, 0], (0, pad))
    idx1 = jnp.pad(idx[:, 1], (0, pad))
    len_p = jnp.pad(edge_len, (0, pad))

    mesh = plsc.VectorSubcoreMesh(core_axis_name="c", subcore_axis_name="s")
    run = pl.kernel(
        _tv_kernel,
        mesh=mesh,
        compiler_params=pltpu.CompilerParams(use_tc_tiling_on_sc=True,
                                             needs_layout_passes=False),
        out_type=jax.ShapeDtypeStruct((NW * L,), jnp.float32),
        scratch_types=[
            pltpu.VMEM((B,), jnp.int32),
            pltpu.VMEM((B,), jnp.int32),
            pltpu.VMEM((B,), jnp.float32),
            pltpu.VMEM((B, 128), jnp.int32),
            pltpu.VMEM((B, 128), jnp.int32),
            pltpu.VMEM((B,), jnp.int32),
            pltpu.VMEM((B,), jnp.int32),
            pltpu.VMEM((B,), jnp.float32),
            pltpu.VMEM((B, 128), jnp.int32),
            pltpu.VMEM((B, 128), jnp.int32),
            pltpu.VMEM((L,), jnp.float32),
            pltpu.SemaphoreType.DMA,
            pltpu.SemaphoreType.DMA,
            pltpu.SemaphoreType.DMA,
        ],
    )
    partials = run(patch_p, idx0, idx1, len_p)
    return jnp.sum(partials) / F
